# Initial kernel scaffold; baseline (speedup 1.0000x reference)
import functools

import jax
import jax.numpy as jnp
from jax import lax
from jax.experimental import pallas as pl
from jax.experimental.pallas import tpu as pltpu
from jax.experimental.pallas import tpu_sc as plsc

H = 9
NC, NS = 2, 16
NW = NC * NS
NEG = -1e30


def _mesh():
    return plsc.VectorSubcoreMesh(core_axis_name="c", subcore_axis_name="s")


def _node_tables(xs, u_pad, w, cout, rows=1000):
    m = xs[0].shape[0]
    nx = len(xs)

    def body(*refs):
        x_refs = refs[:nx]
        u_refs = refs[nx:2 * nx]
        w_refs = refs[2 * nx:3 * nx]
        t_ref, xw_ref = refs[3 * nx:]
        t = jnp.dot(x_refs[0][...], u_refs[0][...],
                    preferred_element_type=jnp.float32)
        xw = jnp.dot(x_refs[0][...], w_refs[0][...],
                     preferred_element_type=jnp.float32)
        for i in range(1, nx):
            t += jnp.dot(x_refs[i][...], u_refs[i][...],
                         preferred_element_type=jnp.float32)
            xw += jnp.dot(x_refs[i][...], w_refs[i][...],
                          preferred_element_type=jnp.float32)
        t_ref[...] = t
        xw_ref[...] = xw

    in_specs = (
        [pl.BlockSpec((rows, x.shape[1]), lambda i: (i, 0)) for x in xs]
        + [pl.BlockSpec(uu.shape, lambda i: (0, 0)) for uu in u_pad]
        + [pl.BlockSpec(ww.shape, lambda i: (0, 0)) for ww in w]
    )
    out_specs = [pl.BlockSpec((rows, 16), lambda i: (i, 0)),
                 pl.BlockSpec((rows, H * cout), lambda i: (i, 0))]
    return pl.pallas_call(
        body,
        grid=(m // rows,),
        in_specs=in_specs,
        out_specs=out_specs,
        out_shape=[jax.ShapeDtypeStruct((m, 16), jnp.float32),
                   jax.ShapeDtypeStruct((m, H * cout), jnp.float32)],
    )(*xs, *u_pad, *w)


def _epilogue(s, cnt, b, lrelu, rows=1000):
    m, cout = s.shape

    def body(s_ref, c_ref, b_ref, o_ref):
        cn = jnp.maximum(c_ref[:, 0:1], 1.0)
        y = s_ref[...] / cn + b_ref[...]
        if lrelu:
            y = jnp.where(y >= 0, y, 0.1 * y)
        o_ref[...] = y

    return pl.pallas_call(
        body,
        grid=(m // rows,),
        in_specs=[pl.BlockSpec((rows, cout), lambda i: (i, 0)),
                  pl.BlockSpec((rows, 16), lambda i: (i, 0)),
                  pl.BlockSpec((1, cout), lambda i: (0, 0))],
        out_specs=pl.BlockSpec((rows, cout), lambda i: (i, 0)),
        out_shape=jax.ShapeDtypeStruct((m, cout), jnp.float32),
    )(s, cnt, b.reshape(1, cout))


def _pair_max(y, rows=1000):
    m, c = y.shape
    y2 = y.reshape(m // 2, 2 * c)

    def body(a_ref, o_ref):
        a = a_ref[...]
        o_ref[...] = jnp.maximum(a[:, :c], a[:, c:])

    return pl.pallas_call(
        body,
        grid=(m // 2 // rows,),
        in_specs=[pl.BlockSpec((rows, 2 * c), lambda i: (i, 0))],
        out_specs=pl.BlockSpec((rows, c), lambda i: (i, 0)),
        out_shape=jax.ShapeDtypeStruct((m // 2, c), jnp.float32),
    )(y2)


def _fc_head(f, w1, b1, w2, b2, rows=1000):
    m = f.shape[0]

    def body(f_ref, w1_ref, b1_ref, w2_ref, b2_ref, o_ref):
        h1 = jnp.dot(f_ref[...], w1_ref[...],
                     preferred_element_type=jnp.float32) + b1_ref[...]
        h1 = jnp.where(h1 >= 0, h1, 0.1 * h1)
        o = jnp.dot(h1, w2_ref[...],
                    preferred_element_type=jnp.float32) + b2_ref[...]
        nrm = jnp.sqrt(jnp.sum(o * o, axis=1, keepdims=True))
        o_ref[...] = o / jnp.maximum(nrm, 1e-12)

    return pl.pallas_call(
        body,
        grid=(m // rows,),
        in_specs=[pl.BlockSpec((rows, f.shape[1]), lambda i: (i, 0)),
                  pl.BlockSpec(w1.shape, lambda i: (0, 0)),
                  pl.BlockSpec((1, b1.shape[0]), lambda i: (0, 0)),
                  pl.BlockSpec(w2.shape, lambda i: (0, 0)),
                  pl.BlockSpec((1, b2.shape[0]), lambda i: (0, 0))],
        out_specs=pl.BlockSpec((rows, w2.shape[1]), lambda i: (i, 0)),
        out_shape=jax.ShapeDtypeStruct((m, w2.shape[1]), jnp.float32),
    )(f, w1, b1.reshape(1, -1), w2, b2.reshape(1, -1))


def _edge_msgs(src0, dst0, t_tab, xw_tab, c16, g, cout, chunk):
    e_tot = src0.shape[0]
    per_tile = e_tot // NW
    n_chunks = per_tile // chunk
    nb = cout // 16
    wrow = H * cout

    @functools.partial(
        pl.kernel,
        out_type=jax.ShapeDtypeStruct((e_tot, cout), jnp.float32),
        mesh=_mesh(),
        scratch_types=[
            pltpu.VMEM((chunk,), jnp.int32),
            pltpu.VMEM((chunk,), jnp.int32),
            pltpu.VMEM((chunk,), jnp.int32),
            pltpu.VMEM((chunk,), jnp.int32),
            pltpu.VMEM((chunk, 16), jnp.float32),
            pltpu.VMEM((chunk, 16), jnp.float32),
            pltpu.VMEM((chunk, wrow), jnp.float32),
            pltpu.VMEM((chunk, cout), jnp.float32),
            pltpu.VMEM((16,), jnp.float32),
            pltpu.VMEM((16,), jnp.float32),
            pltpu.SemaphoreType.DMA,
        ],
    )
    def k(src_h, dst_h, t_h, xw_h, c_h, msg_h,
          rs_v, rd_v, gs_v, gd_v, ts_v, td_v, xw_v, msg_v, c_v, q_v, sem):
        wid = lax.axis_index("s") * NC + lax.axis_index("c")
        base0 = wid * per_tile
        pltpu.sync_copy(c_h, c_v)
        c16v = c_v[...]

        def chunk_body(kk, _):
            base = base0 + kk * chunk
            pltpu.sync_copy(src_h.at[pl.ds(base, chunk)], rs_v)
            pltpu.sync_copy(dst_h.at[pl.ds(base, chunk)], rd_v)

            def shift_body(j, _):
                sl = pl.ds(j * 16, 16)
                gs_v[sl] = lax.shift_right_logical(rs_v[sl], g)
                gd_v[sl] = lax.shift_right_logical(rd_v[sl], g)
                return 0

            lax.fori_loop(0, chunk // 16, shift_body, 0)
            cp1 = pltpu.async_copy(t_h.at[gs_v], ts_v, sem)
            cp2 = pltpu.async_copy(t_h.at[gd_v], td_v, sem)
            cp3 = pltpu.async_copy(xw_h.at[gs_v], xw_v, sem)
            cp1.wait()
            cp2.wait()
            cp3.wait()

            def edge_body(e, _):
                d = ts_v[e, :] - td_v[e, :] + c16v
                mx = jnp.max(d)
                ex = jnp.exp(d - mx)
                q_v[...] = ex / jnp.sum(ex)
                acc = [jnp.zeros((16,), jnp.float32) for _ in range(nb)]
                for hh in range(H):
                    qh = q_v[hh]
                    for cb in range(nb):
                        o = hh * cout + cb * 16
                        acc[cb] = acc[cb] + qh * xw_v[e, o:o + 16]
                for cb in range(nb):
                    msg_v[e, cb * 16:(cb + 1) * 16] = acc[cb]
                return 0

            lax.fori_loop(0, chunk, edge_body, 0)
            pltpu.sync_copy(msg_v, msg_h.at[pl.ds(base, chunk)])
            return 0

        lax.fori_loop(0, n_chunks, chunk_body, 0)

    return k(src0, dst0, t_tab, xw_tab, c16)


def _seg_sum(dst0, msg, d_shift, n, cout, chunk=400):
    e_tot = dst0.shape[0]
    half = n // 2
    rows = half + 100
    per_tile = e_tot // NS
    n_chunks = per_tile // chunk
    zc = rows // 100
    fc = half // 100

    @functools.partial(
        pl.kernel,
        out_type=jax.ShapeDtypeStruct((n, cout), jnp.float32),
        mesh=_mesh(),
        scratch_types=[
            pltpu.VMEM((chunk,), jnp.int32),
            pltpu.VMEM((chunk,), jnp.int32),
            pltpu.VMEM((chunk, cout), jnp.float32),
            pltpu.VMEM((100, cout), jnp.float32),
            pltpu.VMEM_SHARED((rows, cout), jnp.float32),
        ],
    )
    def k(dst_h, msg_h, z_h, out_h, rd_v, idx_v, msg_v, z_v, acc):
        cid = lax.axis_index("c")
        sid = lax.axis_index("s")
        node_base = cid * half
        pltpu.sync_copy(z_h, z_v)
        for zk in range((zc + NS - 1) // NS):
            ck = zk * NS + sid

            @pl.when(ck < zc)
            def _():
                pltpu.sync_copy(z_v, acc.at[pl.ds(ck * 100, 100)])
        plsc.subcore_barrier()

        def chunk_body(kk, _):
            base = sid * per_tile + kk * chunk
            pltpu.sync_copy(dst_h.at[pl.ds(base, chunk)], rd_v)
            pltpu.sync_copy(msg_h.at[pl.ds(base, chunk)], msg_v)

            def shift_body(j, _):
                sl = pl.ds(j * 16, 16)
                loc = lax.shift_right_logical(rd_v[sl], d_shift) - node_base
                ok = (loc >= 0) & (loc < half)
                idx_v[sl] = jnp.where(ok, loc, half + sid)
                return 0

            lax.fori_loop(0, chunk // 16, shift_body, 0)
            pltpu.sync_copy(msg_v, acc.at[idx_v], add=True)
            return 0

        lax.fori_loop(0, n_chunks, chunk_body, 0)
        plsc.subcore_barrier()
        for fk in range((fc + NS - 1) // NS):
            ck = fk * NS + sid

            @pl.when(ck < fc)
            def _():
                pltpu.sync_copy(acc.at[pl.ds(ck * 100, 100)],
                                out_h.at[pl.ds(node_base + ck * 100, 100)])

    zeros = jnp.zeros((100, cout), jnp.float32)
    return k(dst0, msg, zeros)


def _counts(dst0, n1, chunk=400):
    e_tot = dst0.shape[0]
    halves = (n1 // 2, n1 // 4, n1 // 8)
    rows = tuple(h + 100 for h in halves)
    per_tile = e_tot // NS
    n_chunks = per_tile // chunk

    @functools.partial(
        pl.kernel,
        out_type=[jax.ShapeDtypeStruct((n1, 16), jnp.float32),
                  jax.ShapeDtypeStruct((n1 // 2, 16), jnp.float32),
                  jax.ShapeDtypeStruct((n1 // 4, 16), jnp.float32)],
        mesh=_mesh(),
        scratch_types=[
            pltpu.VMEM((chunk,), jnp.int32),
            pltpu.VMEM((chunk,), jnp.int32),
            pltpu.VMEM((chunk, 16), jnp.float32),
            pltpu.VMEM((100, 16), jnp.float32),
            pltpu.VMEM_SHARED((rows[0], 16), jnp.float32),
            pltpu.VMEM_SHARED((rows[1], 16), jnp.float32),
            pltpu.VMEM_SHARED((rows[2], 16), jnp.float32),
        ],
    )
    def k(dst_h, ones_h, z_h, c1_h, c2_h, c3_h,
          rd_v, idx_v, ones_v, z_v, acc1, acc2, acc3):
        cid = lax.axis_index("c")
        sid = lax.axis_index("s")
        pltpu.sync_copy(ones_h, ones_v)
        pltpu.sync_copy(z_h, z_v)
        for acc, rw in ((acc1, rows[0]), (acc2, rows[1]), (acc3, rows[2])):
            zc = rw // 100
            for zk in range((zc + NS - 1) // NS):
                ck = zk * NS + sid

                @pl.when(ck < zc)
                def _():
                    pltpu.sync_copy(z_v, acc.at[pl.ds(ck * 100, 100)])
        plsc.subcore_barrier()

        def chunk_body(kk, _):
            base = sid * per_tile + kk * chunk
            pltpu.sync_copy(dst_h.at[pl.ds(base, chunk)], rd_v)
            for lvl, (acc, hf) in enumerate(((acc1, halves[0]),
                                             (acc2, halves[1]),
                                             (acc3, halves[2]))):
                def shift_body(j, _, lvl=lvl, hf=hf):
                    sl = pl.ds(j * 16, 16)
                    loc = lax.shift_right_logical(rd_v[sl], lvl) - cid * hf
                    ok = (loc >= 0) & (loc < hf)
                    idx_v[sl] = jnp.where(ok, loc, hf + sid)
                    return 0

                lax.fori_loop(0, chunk // 16, shift_body, 0)
                pltpu.sync_copy(ones_v, acc.at[idx_v], add=True)
            return 0

        lax.fori_loop(0, n_chunks, chunk_body, 0)
        plsc.subcore_barrier()
        for acc, hf, out_h in ((acc1, halves[0], c1_h),
                               (acc2, halves[1], c2_h),
                               (acc3, halves[2], c3_h)):
            fc = hf // 100
            for fk in range((fc + NS - 1) // NS):
                ck = fk * NS + sid

                @pl.when(ck < fc)
                def _():
                    pltpu.sync_copy(acc.at[pl.ds(ck * 100, 100)],
                                    out_h.at[pl.ds(cid * hf + ck * 100, 100)])

    ones = jnp.zeros((chunk, 16), jnp.float32).at[:, 0].set(1.0)
    zeros = jnp.zeros((100, 16), jnp.float32)
    return k(dst0, ones, zeros)


def _pad_u(u):
    cin = u.shape[0]
    cp = max(8, cin)
    out = jnp.zeros((cp, 16), jnp.float32)
    return out.at[:cin, :H].set(u)


def _pad_w(w):
    cin = w.shape[0]
    cp = max(8, cin)
    if cp == cin:
        return w
    return jnp.zeros((cp, w.shape[1]), jnp.float32).at[:cin].set(w)


def _c16(c):
    return jnp.concatenate([c, jnp.full((16 - H,), NEG, jnp.float32)])


def _feast(xs, src0, dst0, cnt, p, g, d_shift, n_seg, cout, lrelu, chunk):
    u_parts, w_parts, x_parts = [], [], []
    off = 0
    for xx in xs:
        cw = xx.shape[1]
        u_parts.append(_pad_u(p["u"][off:off + cw]))
        w_parts.append(_pad_w(p["W"][off:off + cw]))
        x_parts.append(xx if cw >= 8
                       else jnp.pad(xx, ((0, 0), (0, 8 - cw))))
        off += cw
    t_tab, xw_tab = _node_tables(x_parts, u_parts, w_parts, cout)
    msg = _edge_msgs(src0, dst0, t_tab, xw_tab, _c16(p["c"]), g, cout, chunk)
    s = _seg_sum(dst0, msg, d_shift, n_seg, cout)
    return _epilogue(s, cnt, p["b"], lrelu)


def kernel(x, edge_index, params):
    p = params
    src0 = edge_index[0]
    dst0 = edge_index[1]
    n1 = x.shape[0]

    cnt1, cnt2, cnt3 = _counts(dst0, n1)

    x1 = _feast([x], src0, dst0, cnt1, p["l1"], 0, 0, n1, 32, True, 200)
    x2p = _pair_max(x1)
    x2 = _feast([x2p], src0, dst0, cnt2, p["l2"], 1, 1, n1 // 2, 64, True, 80)
    x3p = _pair_max(x2)
    x3 = _feast([x3p], src0, dst0, cnt3, p["l3"], 2, 2, n1 // 4, 128, True, 80)
    x3 = _feast([x3], src0, dst0, cnt3, p["l4"], 2, 2, n1 // 4, 128, True, 80)
    f2 = _feast([x3], src0, dst0, cnt2, p["r1"], 2, 1, n1 // 2, 64, False, 80)
    y2 = _feast([x2, f2], src0, dst0, cnt2, p["r2"], 1, 1, n1 // 2, 64, True, 80)
    f1 = _feast([y2], src0, dst0, cnt1, p["r3"], 1, 0, n1, 32, False, 200)
    f = _feast([x1, f1], src0, dst0, cnt1, p["r4"], 0, 0, n1, 32, True, 200)
    return _fc_head(f, p["fc1w"], p["fc1b"], p["fc2w"], p["fc2b"])


# speedup vs baseline: 2.9200x; 2.9200x over previous
import functools

import jax
import jax.numpy as jnp
from jax import lax
from jax.experimental import pallas as pl
from jax.experimental.pallas import tpu as pltpu
from jax.experimental.pallas import tpu_sc as plsc

H = 9
NC, NS = 2, 16
NW = NC * NS
NEG = -1e30


def _mesh():
    return plsc.VectorSubcoreMesh(core_axis_name="c", subcore_axis_name="s")


def _perm(v, idx):
    return v.at[idx].get(mode="promise_in_bounds")


def _lanes_max(v):
    i = lax.iota(jnp.int32, 16)
    for sh in (8, 4, 2, 1):
        v = jnp.maximum(v, _perm(v, i ^ sh))
    return v


def _lanes_sum(v):
    i = lax.iota(jnp.int32, 16)
    for sh in (8, 4, 2, 1):
        v = v + _perm(v, i ^ sh)
    return v


def _node_tables(xs, u_pad, w, cout, rows=1000):
    m = xs[0].shape[0]
    nx = len(xs)

    def body(*refs):
        x_refs = refs[:nx]
        u_refs = refs[nx:2 * nx]
        w_refs = refs[2 * nx:3 * nx]
        t_ref, xw_ref = refs[3 * nx:]
        t = jnp.dot(x_refs[0][...], u_refs[0][...],
                    preferred_element_type=jnp.float32)
        xw = jnp.dot(x_refs[0][...], w_refs[0][...],
                     preferred_element_type=jnp.float32)
        for i in range(1, nx):
            t += jnp.dot(x_refs[i][...], u_refs[i][...],
                         preferred_element_type=jnp.float32)
            xw += jnp.dot(x_refs[i][...], w_refs[i][...],
                          preferred_element_type=jnp.float32)
        t_ref[...] = t
        xw_ref[...] = xw

    in_specs = (
        [pl.BlockSpec((rows, x.shape[1]), lambda i: (i, 0)) for x in xs]
        + [pl.BlockSpec(uu.shape, lambda i: (0, 0)) for uu in u_pad]
        + [pl.BlockSpec(ww.shape, lambda i: (0, 0)) for ww in w]
    )
    out_specs = [pl.BlockSpec((rows, 16), lambda i: (i, 0)),
                 pl.BlockSpec((rows, H * cout), lambda i: (i, 0))]
    return pl.pallas_call(
        body,
        grid=(m // rows,),
        in_specs=in_specs,
        out_specs=out_specs,
        out_shape=[jax.ShapeDtypeStruct((m, 16), jnp.float32),
                   jax.ShapeDtypeStruct((m, H * cout), jnp.float32)],
    )(*xs, *u_pad, *w)


def _epilogue(s, cnt, b, lrelu, rows=1000):
    m, cout = s.shape

    def body(s_ref, c_ref, b_ref, o_ref):
        cn = jnp.maximum(c_ref[:, 0:1], 1.0)
        y = s_ref[...] / cn + b_ref[...]
        if lrelu:
            y = jnp.where(y >= 0, y, 0.1 * y)
        o_ref[...] = y

    return pl.pallas_call(
        body,
        grid=(m // rows,),
        in_specs=[pl.BlockSpec((rows, cout), lambda i: (i, 0)),
                  pl.BlockSpec((rows, 16), lambda i: (i, 0)),
                  pl.BlockSpec((1, cout), lambda i: (0, 0))],
        out_specs=pl.BlockSpec((rows, cout), lambda i: (i, 0)),
        out_shape=jax.ShapeDtypeStruct((m, cout), jnp.float32),
    )(s, cnt, b.reshape(1, cout))


def _pair_max(y, rows=1000):
    m, c = y.shape
    y2 = y.reshape(m // 2, 2 * c)

    def body(a_ref, o_ref):
        a = a_ref[...]
        o_ref[...] = jnp.maximum(a[:, :c], a[:, c:])

    return pl.pallas_call(
        body,
        grid=(m // 2 // rows,),
        in_specs=[pl.BlockSpec((rows, 2 * c), lambda i: (i, 0))],
        out_specs=pl.BlockSpec((rows, c), lambda i: (i, 0)),
        out_shape=jax.ShapeDtypeStruct((m // 2, c), jnp.float32),
    )(y2)


def _fc_head(f, w1, b1, w2, b2, rows=1000):
    m = f.shape[0]

    def body(f_ref, w1_ref, b1_ref, w2_ref, b2_ref, o_ref):
        h1 = jnp.dot(f_ref[...], w1_ref[...],
                     preferred_element_type=jnp.float32) + b1_ref[...]
        h1 = jnp.where(h1 >= 0, h1, 0.1 * h1)
        o = jnp.dot(h1, w2_ref[...],
                    preferred_element_type=jnp.float32) + b2_ref[...]
        nrm = jnp.sqrt(jnp.sum(o * o, axis=1, keepdims=True))
        o_ref[...] = o / jnp.maximum(nrm, 1e-12)

    return pl.pallas_call(
        body,
        grid=(m // rows,),
        in_specs=[pl.BlockSpec((rows, f.shape[1]), lambda i: (i, 0)),
                  pl.BlockSpec(w1.shape, lambda i: (0, 0)),
                  pl.BlockSpec((1, b1.shape[0]), lambda i: (0, 0)),
                  pl.BlockSpec(w2.shape, lambda i: (0, 0)),
                  pl.BlockSpec((1, b2.shape[0]), lambda i: (0, 0))],
        out_specs=pl.BlockSpec((rows, w2.shape[1]), lambda i: (i, 0)),
        out_shape=jax.ShapeDtypeStruct((m, w2.shape[1]), jnp.float32),
    )(f, w1, b1.reshape(1, -1), w2, b2.reshape(1, -1))


def _edge_msgs(src0, dst0, t_tab, xw_tab, c16, g, cout, chunk):
    e_tot = src0.shape[0]
    per_tile = e_tot // NW
    n_chunks = per_tile // chunk
    nb = cout // 16
    wrow = H * cout

    @functools.partial(
        pl.kernel,
        out_type=jax.ShapeDtypeStruct((e_tot, cout), jnp.float32),
        mesh=_mesh(),
        compiler_params=pltpu.CompilerParams(use_tc_tiling_on_sc=False),
        scratch_types=[
            pltpu.VMEM((chunk,), jnp.int32),
            pltpu.VMEM((chunk,), jnp.int32),
            pltpu.VMEM((chunk,), jnp.int32),
            pltpu.VMEM((chunk,), jnp.int32),
            pltpu.VMEM((chunk, 16), jnp.float32),
            pltpu.VMEM((chunk, 16), jnp.float32),
            pltpu.VMEM((chunk, wrow), jnp.float32),
            pltpu.VMEM((chunk, cout), jnp.float32),
            pltpu.VMEM((16,), jnp.float32),
            pltpu.SemaphoreType.DMA,
        ],
    )
    def k(src_h, dst_h, t_h, xw_h, c_h, msg_h,
          rs_v, rd_v, gs_v, gd_v, ts_v, td_v, xw_v, msg_v, c_v, sem):
        wid = lax.axis_index("s") * NC + lax.axis_index("c")
        base0 = wid * per_tile
        pltpu.sync_copy(c_h, c_v)
        c16v = c_v[...]

        def chunk_body(kk, _):
            base = base0 + kk * chunk
            pltpu.sync_copy(src_h.at[pl.ds(base, chunk)], rs_v)
            pltpu.sync_copy(dst_h.at[pl.ds(base, chunk)], rd_v)

            def shift_body(j, _):
                sl = pl.ds(j * 16, 16)
                gs_v[sl] = lax.shift_right_logical(rs_v[sl], g)
                gd_v[sl] = lax.shift_right_logical(rd_v[sl], g)
                return 0

            lax.fori_loop(0, chunk // 16, shift_body, 0)
            cp1 = pltpu.async_copy(t_h.at[gs_v], ts_v, sem)
            cp2 = pltpu.async_copy(t_h.at[gd_v], td_v, sem)
            cp3 = pltpu.async_copy(xw_h.at[gs_v], xw_v, sem)
            cp1.wait()
            cp2.wait()
            cp3.wait()

            def edge_body(e, _):
                d = ts_v[e, :] - td_v[e, :] + c16v
                ex = jnp.exp(d - _lanes_max(d))
                q = ex / _lanes_sum(ex)
                acc = [jnp.zeros((16,), jnp.float32) for _ in range(nb)]
                for hh in range(H):
                    qh = _perm(q, jnp.full((16,), hh, jnp.int32))
                    for cb in range(nb):
                        o = hh * cout + cb * 16
                        acc[cb] = acc[cb] + qh * xw_v[e, o:o + 16]
                for cb in range(nb):
                    msg_v[e, cb * 16:(cb + 1) * 16] = acc[cb]
                return 0

            lax.fori_loop(0, chunk, edge_body, 0)
            pltpu.sync_copy(msg_v, msg_h.at[pl.ds(base, chunk)])
            return 0

        lax.fori_loop(0, n_chunks, chunk_body, 0)

    return k(src0, dst0, t_tab, xw_tab, c16)


def _seg_sum(dst0, msg, d_shift, n, cout, chunk=None):
    if chunk is None:
        chunk = 80
    e_tot = dst0.shape[0]
    half = n // 2
    rows = half + 100
    per_tile = e_tot // NS
    n_chunks = per_tile // chunk
    zc = rows // 100
    fc = half // 100

    @functools.partial(
        pl.kernel,
        out_type=jax.ShapeDtypeStruct((n, cout), jnp.float32),
        mesh=_mesh(),
        compiler_params=pltpu.CompilerParams(use_tc_tiling_on_sc=False),
        scratch_types=[
            pltpu.VMEM((chunk,), jnp.int32),
            pltpu.VMEM((chunk,), jnp.int32),
            pltpu.VMEM((chunk, cout), jnp.float32),
            pltpu.VMEM_SHARED((rows, cout), jnp.float32),
        ],
    )
    def k(dst_h, msg_h, z_h, out_h, rd_v, idx_v, msg_v, acc):
        cid = lax.axis_index("c")
        sid = lax.axis_index("s")
        node_base = cid * half
        for zk in range((zc + NS - 1) // NS):
            ck = zk * NS + sid

            @pl.when(ck < zc)
            def _():
                pltpu.sync_copy(z_h, acc.at[pl.ds(ck * 100, 100)])
        plsc.subcore_barrier()

        def chunk_body(kk, _):
            base = sid * per_tile + kk * chunk
            pltpu.sync_copy(dst_h.at[pl.ds(base, chunk)], rd_v)
            pltpu.sync_copy(msg_h.at[pl.ds(base, chunk)], msg_v)

            def shift_body(j, _):
                sl = pl.ds(j * 16, 16)
                loc = lax.shift_right_logical(rd_v[sl], d_shift) - node_base
                ok = (loc >= 0) & (loc < half)
                idx_v[sl] = jnp.where(ok, loc, half + sid)
                return 0

            lax.fori_loop(0, chunk // 16, shift_body, 0)
            pltpu.sync_copy(msg_v, acc.at[idx_v], add=True)
            return 0

        lax.fori_loop(0, n_chunks, chunk_body, 0)
        plsc.subcore_barrier()
        for fk in range((fc + NS - 1) // NS):
            ck = fk * NS + sid

            @pl.when(ck < fc)
            def _():
                pltpu.sync_copy(acc.at[pl.ds(ck * 100, 100)],
                                out_h.at[pl.ds(node_base + ck * 100, 100)])

    zeros = jnp.zeros((100, cout), jnp.float32)
    return k(dst0, msg, zeros)


def _counts(dst0, n1, chunk=80):
    e_tot = dst0.shape[0]
    halves = (n1 // 2, n1 // 4, n1 // 8)
    rows = tuple(h + 100 for h in halves)
    per_tile = e_tot // NS
    n_chunks = per_tile // chunk

    @functools.partial(
        pl.kernel,
        out_type=[jax.ShapeDtypeStruct((n1, 16), jnp.float32),
                  jax.ShapeDtypeStruct((n1 // 2, 16), jnp.float32),
                  jax.ShapeDtypeStruct((n1 // 4, 16), jnp.float32)],
        mesh=_mesh(),
        compiler_params=pltpu.CompilerParams(use_tc_tiling_on_sc=False),
        scratch_types=[
            pltpu.VMEM((chunk,), jnp.int32),
            pltpu.VMEM((chunk,), jnp.int32),
            pltpu.VMEM((chunk, 16), jnp.float32),
            pltpu.VMEM((100, 16), jnp.float32),
            pltpu.VMEM_SHARED((rows[0], 16), jnp.float32),
            pltpu.VMEM_SHARED((rows[1], 16), jnp.float32),
            pltpu.VMEM_SHARED((rows[2], 16), jnp.float32),
        ],
    )
    def k(dst_h, ones_h, z_h, c1_h, c2_h, c3_h,
          rd_v, idx_v, ones_v, z_v, acc1, acc2, acc3):
        cid = lax.axis_index("c")
        sid = lax.axis_index("s")
        pltpu.sync_copy(ones_h, ones_v)
        pltpu.sync_copy(z_h, z_v)
        for acc, rw in ((acc1, rows[0]), (acc2, rows[1]), (acc3, rows[2])):
            zc = rw // 100
            for zk in range((zc + NS - 1) // NS):
                ck = zk * NS + sid

                @pl.when(ck < zc)
                def _():
                    pltpu.sync_copy(z_v, acc.at[pl.ds(ck * 100, 100)])
        plsc.subcore_barrier()

        def chunk_body(kk, _):
            base = sid * per_tile + kk * chunk
            pltpu.sync_copy(dst_h.at[pl.ds(base, chunk)], rd_v)
            for lvl, (acc, hf) in enumerate(((acc1, halves[0]),
                                             (acc2, halves[1]),
                                             (acc3, halves[2]))):
                def shift_body(j, _, lvl=lvl, hf=hf):
                    sl = pl.ds(j * 16, 16)
                    loc = lax.shift_right_logical(rd_v[sl], lvl) - cid * hf
                    ok = (loc >= 0) & (loc < hf)
                    idx_v[sl] = jnp.where(ok, loc, hf + sid)
                    return 0

                lax.fori_loop(0, chunk // 16, shift_body, 0)
                pltpu.sync_copy(ones_v, acc.at[idx_v], add=True)
            return 0

        lax.fori_loop(0, n_chunks, chunk_body, 0)
        plsc.subcore_barrier()
        for acc, hf, out_h in ((acc1, halves[0], c1_h),
                               (acc2, halves[1], c2_h),
                               (acc3, halves[2], c3_h)):
            fc = hf // 100
            for fk in range((fc + NS - 1) // NS):
                ck = fk * NS + sid

                @pl.when(ck < fc)
                def _():
                    pltpu.sync_copy(acc.at[pl.ds(ck * 100, 100)],
                                    out_h.at[pl.ds(cid * hf + ck * 100, 100)])

    ones = jnp.zeros((chunk, 16), jnp.float32).at[:, 0].set(1.0)
    zeros = jnp.zeros((100, 16), jnp.float32)
    return k(dst0, ones, zeros)


def _pad_u(u):
    cin = u.shape[0]
    cp = max(8, cin)
    out = jnp.zeros((cp, 16), jnp.float32)
    return out.at[:cin, :H].set(u)


def _pad_w(w):
    cin = w.shape[0]
    cp = max(8, cin)
    if cp == cin:
        return w
    return jnp.zeros((cp, w.shape[1]), jnp.float32).at[:cin].set(w)


def _c16(c):
    return jnp.concatenate([c, jnp.full((16 - H,), NEG, jnp.float32)])


def _feast(xs, src0, dst0, cnt, p, g, d_shift, n_seg, cout, lrelu, chunk):
    u_parts, w_parts, x_parts = [], [], []
    off = 0
    for xx in xs:
        cw = xx.shape[1]
        u_parts.append(_pad_u(p["u"][off:off + cw]))
        w_parts.append(_pad_w(p["W"][off:off + cw]))
        x_parts.append(xx if cw >= 8
                       else jnp.pad(xx, ((0, 0), (0, 8 - cw))))
        off += cw
    t_tab, xw_tab = _node_tables(x_parts, u_parts, w_parts, cout)
    msg = _edge_msgs(src0, dst0, t_tab, xw_tab, _c16(p["c"]), g, cout, chunk)
    s = _seg_sum(dst0, msg, d_shift, n_seg, cout)
    return _epilogue(s, cnt, p["b"], lrelu)


def kernel(x, edge_index, params):
    p = params
    src0 = edge_index[0]
    dst0 = edge_index[1]
    n1 = x.shape[0]

    cnt1, cnt2, cnt3 = _counts(dst0, n1)

    x1 = _feast([x], src0, dst0, cnt1, p["l1"], 0, 0, n1, 32, True, 80)
    x2p = _pair_max(x1)
    x2 = _feast([x2p], src0, dst0, cnt2, p["l2"], 1, 1, n1 // 2, 64, True, 80)
    x3p = _pair_max(x2)
    x3 = _feast([x3p], src0, dst0, cnt3, p["l3"], 2, 2, n1 // 4, 128, True, 80)
    x3 = _feast([x3], src0, dst0, cnt3, p["l4"], 2, 2, n1 // 4, 128, True, 80)
    f2 = _feast([x3], src0, dst0, cnt2, p["r1"], 2, 1, n1 // 2, 64, False, 80)
    y2 = _feast([x2, f2], src0, dst0, cnt2, p["r2"], 1, 1, n1 // 2, 64, True, 80)
    f1 = _feast([y2], src0, dst0, cnt1, p["r3"], 1, 0, n1, 32, False, 80)
    f = _feast([x1, f1], src0, dst0, cnt1, p["r4"], 0, 0, n1, 32, True, 80)
    return _fc_head(f, p["fc1w"], p["fc1b"], p["fc2w"], p["fc2b"])


# speedup vs baseline: 4.2716x; 1.4629x over previous
import functools

import jax
import jax.numpy as jnp
from jax import lax
from jax.experimental import pallas as pl
from jax.experimental.pallas import tpu as pltpu
from jax.experimental.pallas import tpu_sc as plsc

H = 9
NC, NS = 2, 16
NW = NC * NS
NEG = -1e30


def _mesh():
    return plsc.VectorSubcoreMesh(core_axis_name="c", subcore_axis_name="s")


def _perm(v, idx):
    return v.at[idx].get(mode="promise_in_bounds")


def _lanes_max(v):
    i = lax.iota(jnp.int32, 16)
    for sh in (8, 4, 2, 1):
        v = jnp.maximum(v, _perm(v, i ^ sh))
    return v


def _lanes_sum(v):
    i = lax.iota(jnp.int32, 16)
    for sh in (8, 4, 2, 1):
        v = v + _perm(v, i ^ sh)
    return v


def _node_tables(xs, u_pad, w, cout, rows=1000):
    m = xs[0].shape[0]
    nx = len(xs)

    def body(*refs):
        x_refs = refs[:nx]
        u_refs = refs[nx:2 * nx]
        w_refs = refs[2 * nx:3 * nx]
        t_ref, xw_ref = refs[3 * nx:]
        t = jnp.dot(x_refs[0][...], u_refs[0][...],
                    preferred_element_type=jnp.float32)
        xw = jnp.dot(x_refs[0][...], w_refs[0][...],
                     preferred_element_type=jnp.float32)
        for i in range(1, nx):
            t += jnp.dot(x_refs[i][...], u_refs[i][...],
                         preferred_element_type=jnp.float32)
            xw += jnp.dot(x_refs[i][...], w_refs[i][...],
                          preferred_element_type=jnp.float32)
        t_ref[...] = t
        xw_ref[...] = xw

    in_specs = (
        [pl.BlockSpec((rows, x.shape[1]), lambda i: (i, 0)) for x in xs]
        + [pl.BlockSpec(uu.shape, lambda i: (0, 0)) for uu in u_pad]
        + [pl.BlockSpec(ww.shape, lambda i: (0, 0)) for ww in w]
    )
    out_specs = [pl.BlockSpec((rows, 16), lambda i: (i, 0)),
                 pl.BlockSpec((rows, H * cout), lambda i: (i, 0))]
    return pl.pallas_call(
        body,
        grid=(m // rows,),
        in_specs=in_specs,
        out_specs=out_specs,
        out_shape=[jax.ShapeDtypeStruct((m, 16), jnp.float32),
                   jax.ShapeDtypeStruct((m, H * cout), jnp.float32)],
    )(*xs, *u_pad, *w)


def _epilogue(s, cnt, b, lrelu, rows=1000):
    m, cout = s.shape

    def body(s_ref, c_ref, b_ref, o_ref):
        cn = jnp.maximum(c_ref[:, 0:1], 1.0)
        y = s_ref[...] / cn + b_ref[...]
        if lrelu:
            y = jnp.where(y >= 0, y, 0.1 * y)
        o_ref[...] = y

    return pl.pallas_call(
        body,
        grid=(m // rows,),
        in_specs=[pl.BlockSpec((rows, cout), lambda i: (i, 0)),
                  pl.BlockSpec((rows, 16), lambda i: (i, 0)),
                  pl.BlockSpec((1, cout), lambda i: (0, 0))],
        out_specs=pl.BlockSpec((rows, cout), lambda i: (i, 0)),
        out_shape=jax.ShapeDtypeStruct((m, cout), jnp.float32),
    )(s, cnt, b.reshape(1, cout))


def _pair_max(y, rows=1000):
    m, c = y.shape
    y2 = y.reshape(m // 2, 2 * c)

    def body(a_ref, o_ref):
        a = a_ref[...]
        o_ref[...] = jnp.maximum(a[:, :c], a[:, c:])

    return pl.pallas_call(
        body,
        grid=(m // 2 // rows,),
        in_specs=[pl.BlockSpec((rows, 2 * c), lambda i: (i, 0))],
        out_specs=pl.BlockSpec((rows, c), lambda i: (i, 0)),
        out_shape=jax.ShapeDtypeStruct((m // 2, c), jnp.float32),
    )(y2)


def _fc_head(f, w1, b1, w2, b2, rows=1000):
    m = f.shape[0]

    def body(f_ref, w1_ref, b1_ref, w2_ref, b2_ref, o_ref):
        h1 = jnp.dot(f_ref[...], w1_ref[...],
                     preferred_element_type=jnp.float32) + b1_ref[...]
        h1 = jnp.where(h1 >= 0, h1, 0.1 * h1)
        o = jnp.dot(h1, w2_ref[...],
                    preferred_element_type=jnp.float32) + b2_ref[...]
        nrm = jnp.sqrt(jnp.sum(o * o, axis=1, keepdims=True))
        o_ref[...] = o / jnp.maximum(nrm, 1e-12)

    return pl.pallas_call(
        body,
        grid=(m // rows,),
        in_specs=[pl.BlockSpec((rows, f.shape[1]), lambda i: (i, 0)),
                  pl.BlockSpec(w1.shape, lambda i: (0, 0)),
                  pl.BlockSpec((1, b1.shape[0]), lambda i: (0, 0)),
                  pl.BlockSpec(w2.shape, lambda i: (0, 0)),
                  pl.BlockSpec((1, b2.shape[0]), lambda i: (0, 0))],
        out_specs=pl.BlockSpec((rows, w2.shape[1]), lambda i: (i, 0)),
        out_shape=jax.ShapeDtypeStruct((m, w2.shape[1]), jnp.float32),
    )(f, w1, b1.reshape(1, -1), w2, b2.reshape(1, -1))


def _shifted_indices(idx, shifts, cols=2000):
    e_tot = idx.shape[0]
    idx2 = idx.reshape(e_tot // cols, cols)

    def body(i_ref, *outs):
        v = i_ref[...]
        for s, o_ref in zip(shifts, outs):
            o_ref[...] = lax.shift_right_logical(v, s)

    rows = 80
    outs = pl.pallas_call(
        body,
        grid=(e_tot // cols // rows,),
        in_specs=[pl.BlockSpec((rows, cols), lambda i: (i, 0))],
        out_specs=[pl.BlockSpec((rows, cols), lambda i: (i, 0))
                   for _ in shifts],
        out_shape=[jax.ShapeDtypeStruct(idx2.shape, jnp.int32)
                   for _ in shifts],
    )(idx2)
    return [o.reshape(e_tot) for o in outs]


def _edge_msgs(gs_arr, gd_arr, t_tab, xw_tab, c16, cout, chunk):
    e_tot = gs_arr.shape[0]
    per_tile = e_tot // NW
    n_chunks = per_tile // chunk
    nb = cout // 16
    wrow = H * cout

    def dbuf(shape, dt):
        return [pltpu.VMEM(shape, dt), pltpu.VMEM(shape, dt)]

    @functools.partial(
        pl.kernel,
        out_type=jax.ShapeDtypeStruct((e_tot, cout), jnp.float32),
        mesh=_mesh(),
        compiler_params=pltpu.CompilerParams(use_tc_tiling_on_sc=False),
        scratch_types=[
            dbuf((chunk,), jnp.int32),
            dbuf((chunk,), jnp.int32),
            dbuf((chunk, 16), jnp.float32),
            dbuf((chunk, 16), jnp.float32),
            dbuf((chunk, wrow), jnp.float32),
            dbuf((chunk, cout), jnp.float32),
            pltpu.VMEM((16,), jnp.float32),
            [pltpu.SemaphoreType.DMA, pltpu.SemaphoreType.DMA],
        ],
    )
    def k(gs_h, gd_h, t_h, xw_h, c_h, msg_h,
          gs_v, gd_v, ts_v, td_v, xw_v, msg_v, c_v, sem):
        wid = lax.axis_index("s") * NC + lax.axis_index("c")
        base0 = wid * per_tile
        pltpu.sync_copy(c_h, c_v)
        c16v = c_v[...]

        def prefetch(c, b):
            base = base0 + c * chunk
            pltpu.sync_copy(gs_h.at[pl.ds(base, chunk)], gs_v[b])
            pltpu.sync_copy(gd_h.at[pl.ds(base, chunk)], gd_v[b])
            pltpu.async_copy(t_h.at[gs_v[b]], ts_v[b], sem[b])
            pltpu.async_copy(t_h.at[gd_v[b]], td_v[b], sem[b])
            pltpu.async_copy(xw_h.at[gs_v[b]], xw_v[b], sem[b])

        def compute(c, b):
            pltpu.make_async_copy(t_h.at[gs_v[b]], ts_v[b], sem[b]).wait()
            pltpu.make_async_copy(t_h.at[gd_v[b]], td_v[b], sem[b]).wait()
            pltpu.make_async_copy(xw_h.at[gs_v[b]], xw_v[b], sem[b]).wait()

            def edge_body(e, _):
                d = ts_v[b][e, :] - td_v[b][e, :] + c16v
                ex = jnp.exp(d - _lanes_max(d))
                q = ex / _lanes_sum(ex)
                acc = [jnp.zeros((16,), jnp.float32) for _ in range(nb)]
                for hh in range(H):
                    qh = _perm(q, jnp.full((16,), hh, jnp.int32))
                    for cb in range(nb):
                        o = hh * cout + cb * 16
                        acc[cb] = acc[cb] + qh * xw_v[b][e, o:o + 16]
                for cb in range(nb):
                    msg_v[b][e, cb * 16:(cb + 1) * 16] = acc[cb]
                return 0

            lax.fori_loop(0, chunk, edge_body, 0)
            pltpu.sync_copy(msg_v[b], msg_h.at[pl.ds(base0 + c * chunk, chunk)])

        prefetch(0, 0)

        def pair_body(kk, _):
            c0 = 2 * kk
            prefetch(c0 + 1, 1)
            compute(c0, 0)

            @pl.when(c0 + 2 < n_chunks)
            def _():
                prefetch(c0 + 2, 0)

            compute(c0 + 1, 1)
            return 0

        lax.fori_loop(0, n_chunks // 2, pair_body, 0)
        if n_chunks % 2 == 1:
            compute(n_chunks - 1, 0)

    return k(gs_arr, gd_arr, t_tab, xw_tab, c16)


def _seg_sum(dst0, msg, d_shift, n, cout, chunk=80):
    e_tot = dst0.shape[0]
    half = n // 2
    rows = half + 100
    per_tile = e_tot // NS
    n_chunks = per_tile // chunk
    zc = rows // 100
    fc = half // 100

    def dbuf(shape, dt):
        return [pltpu.VMEM(shape, dt), pltpu.VMEM(shape, dt)]

    @functools.partial(
        pl.kernel,
        out_type=jax.ShapeDtypeStruct((n, cout), jnp.float32),
        mesh=_mesh(),
        compiler_params=pltpu.CompilerParams(use_tc_tiling_on_sc=False),
        scratch_types=[
            dbuf((chunk,), jnp.int32),
            dbuf((chunk,), jnp.int32),
            dbuf((chunk, cout), jnp.float32),
            pltpu.VMEM_SHARED((rows, cout), jnp.float32),
            [pltpu.SemaphoreType.DMA, pltpu.SemaphoreType.DMA],
        ],
    )
    def k(dst_h, msg_h, z_h, out_h, rd_v, idx_v, msg_v, acc, sem):
        cid = lax.axis_index("c")
        sid = lax.axis_index("s")
        node_base = cid * half
        base0 = sid * per_tile
        for zk in range((zc + NS - 1) // NS):
            ck = zk * NS + sid

            @pl.when(ck < zc)
            def _():
                pltpu.sync_copy(z_h, acc.at[pl.ds(ck * 100, 100)])
        plsc.subcore_barrier()

        def prefetch(c, b):
            base = base0 + c * chunk
            pltpu.sync_copy(dst_h.at[pl.ds(base, chunk)], rd_v[b])
            pltpu.async_copy(msg_h.at[pl.ds(base, chunk)], msg_v[b], sem[b])

        def consume(c, b):
            pltpu.make_async_copy(
                msg_h.at[pl.ds(base0 + c * chunk, chunk)], msg_v[b],
                sem[b]).wait()

            def shift_body(j, _):
                sl = pl.ds(j * 16, 16)
                loc = lax.shift_right_logical(rd_v[b][sl], d_shift) - node_base
                ok = (loc >= 0) & (loc < half)
                idx_v[b][sl] = jnp.where(ok, loc, half + sid)
                return 0

            lax.fori_loop(0, chunk // 16, shift_body, 0)
            pltpu.sync_copy(msg_v[b], acc.at[idx_v[b]], add=True)

        prefetch(0, 0)

        def pair_body(kk, _):
            c0 = 2 * kk
            prefetch(c0 + 1, 1)
            consume(c0, 0)

            @pl.when(c0 + 2 < n_chunks)
            def _():
                prefetch(c0 + 2, 0)

            consume(c0 + 1, 1)
            return 0

        lax.fori_loop(0, n_chunks // 2, pair_body, 0)
        if n_chunks % 2 == 1:
            consume(n_chunks - 1, 0)
        plsc.subcore_barrier()
        for fk in range((fc + NS - 1) // NS):
            ck = fk * NS + sid

            @pl.when(ck < fc)
            def _():
                pltpu.sync_copy(acc.at[pl.ds(ck * 100, 100)],
                                out_h.at[pl.ds(node_base + ck * 100, 100)])

    zeros = jnp.zeros((100, cout), jnp.float32)
    return k(dst0, msg, zeros)


def _counts(dst0, n1, chunk=80):
    e_tot = dst0.shape[0]
    halves = (n1 // 2, n1 // 4, n1 // 8)
    rows = tuple(h + 100 for h in halves)
    per_tile = e_tot // NS
    n_chunks = per_tile // chunk

    @functools.partial(
        pl.kernel,
        out_type=[jax.ShapeDtypeStruct((n1, 16), jnp.float32),
                  jax.ShapeDtypeStruct((n1 // 2, 16), jnp.float32),
                  jax.ShapeDtypeStruct((n1 // 4, 16), jnp.float32)],
        mesh=_mesh(),
        compiler_params=pltpu.CompilerParams(use_tc_tiling_on_sc=False),
        scratch_types=[
            pltpu.VMEM((chunk,), jnp.int32),
            pltpu.VMEM((chunk,), jnp.int32),
            pltpu.VMEM((chunk, 16), jnp.float32),
            pltpu.VMEM((100, 16), jnp.float32),
            pltpu.VMEM_SHARED((rows[0], 16), jnp.float32),
            pltpu.VMEM_SHARED((rows[1], 16), jnp.float32),
            pltpu.VMEM_SHARED((rows[2], 16), jnp.float32),
        ],
    )
    def k(dst_h, ones_h, z_h, c1_h, c2_h, c3_h,
          rd_v, idx_v, ones_v, z_v, acc1, acc2, acc3):
        cid = lax.axis_index("c")
        sid = lax.axis_index("s")
        pltpu.sync_copy(ones_h, ones_v)
        pltpu.sync_copy(z_h, z_v)
        for acc, rw in ((acc1, rows[0]), (acc2, rows[1]), (acc3, rows[2])):
            zc = rw // 100
            for zk in range((zc + NS - 1) // NS):
                ck = zk * NS + sid

                @pl.when(ck < zc)
                def _():
                    pltpu.sync_copy(z_v, acc.at[pl.ds(ck * 100, 100)])
        plsc.subcore_barrier()

        def chunk_body(kk, _):
            base = sid * per_tile + kk * chunk
            pltpu.sync_copy(dst_h.at[pl.ds(base, chunk)], rd_v)
            for lvl, (acc, hf) in enumerate(((acc1, halves[0]),
                                             (acc2, halves[1]),
                                             (acc3, halves[2]))):
                def shift_body(j, _, lvl=lvl, hf=hf):
                    sl = pl.ds(j * 16, 16)
                    loc = lax.shift_right_logical(rd_v[sl], lvl) - cid * hf
                    ok = (loc >= 0) & (loc < hf)
                    idx_v[sl] = jnp.where(ok, loc, hf + sid)
                    return 0

                lax.fori_loop(0, chunk // 16, shift_body, 0)
                pltpu.sync_copy(ones_v, acc.at[idx_v], add=True)
            return 0

        lax.fori_loop(0, n_chunks, chunk_body, 0)
        plsc.subcore_barrier()
        for acc, hf, out_h in ((acc1, halves[0], c1_h),
                               (acc2, halves[1], c2_h),
                               (acc3, halves[2], c3_h)):
            fc = hf // 100
            for fk in range((fc + NS - 1) // NS):
                ck = fk * NS + sid

                @pl.when(ck < fc)
                def _():
                    pltpu.sync_copy(acc.at[pl.ds(ck * 100, 100)],
                                    out_h.at[pl.ds(cid * hf + ck * 100, 100)])

    ones = jnp.zeros((chunk, 16), jnp.float32).at[:, 0].set(1.0)
    zeros = jnp.zeros((100, 16), jnp.float32)
    return k(dst0, ones, zeros)


def _pad_u(u):
    cin = u.shape[0]
    cp = max(8, cin)
    out = jnp.zeros((cp, 16), jnp.float32)
    return out.at[:cin, :H].set(u)


def _pad_w(w):
    cin = w.shape[0]
    cp = max(8, cin)
    if cp == cin:
        return w
    return jnp.zeros((cp, w.shape[1]), jnp.float32).at[:cin].set(w)


def _c16(c):
    return jnp.concatenate([c, jnp.full((16 - H,), NEG, jnp.float32)])


def _feast(xs, gs_arr, gd_arr, dst0, cnt, p, d_shift, n_seg, cout, lrelu):
    u_parts, w_parts, x_parts = [], [], []
    off = 0
    for xx in xs:
        cw = xx.shape[1]
        u_parts.append(_pad_u(p["u"][off:off + cw]))
        w_parts.append(_pad_w(p["W"][off:off + cw]))
        x_parts.append(xx if cw >= 8
                       else jnp.pad(xx, ((0, 0), (0, 8 - cw))))
        off += cw
    t_tab, xw_tab = _node_tables(x_parts, u_parts, w_parts, cout)
    chunk = 40 if cout >= 128 else 80
    msg = _edge_msgs(gs_arr, gd_arr, t_tab, xw_tab, _c16(p["c"]), cout, chunk)
    s = _seg_sum(dst0, msg, d_shift, n_seg, cout)
    return _epilogue(s, cnt, p["b"], lrelu)


def kernel(x, edge_index, params):
    p = params
    src0 = edge_index[0]
    dst0 = edge_index[1]
    n1 = x.shape[0]

    s1, s2 = _shifted_indices(src0, (1, 2))
    d1, d2 = _shifted_indices(dst0, (1, 2))
    cnt1, cnt2, cnt3 = _counts(dst0, n1)

    x1 = _feast([x], src0, dst0, dst0, cnt1, p["l1"], 0, n1, 32, True)
    x2p = _pair_max(x1)
    x2 = _feast([x2p], s1, d1, dst0, cnt2, p["l2"], 1, n1 // 2, 64, True)
    x3p = _pair_max(x2)
    x3 = _feast([x3p], s2, d2, dst0, cnt3, p["l3"], 2, n1 // 4, 128, True)
    x3 = _feast([x3], s2, d2, dst0, cnt3, p["l4"], 2, n1 // 4, 128, True)
    f2 = _feast([x3], s2, d2, dst0, cnt2, p["r1"], 1, n1 // 2, 64, False)
    y2 = _feast([x2, f2], s1, d1, dst0, cnt2, p["r2"], 1, n1 // 2, 64, True)
    f1 = _feast([y2], s1, d1, dst0, cnt1, p["r3"], 0, n1, 32, False)
    f = _feast([x1, f1], src0, dst0, dst0, cnt1, p["r4"], 0, n1, 32, True)
    return _fc_head(f, p["fc1w"], p["fc1b"], p["fc2w"], p["fc2b"])


# speedup vs baseline: 4.3430x; 1.0167x over previous
import functools

import jax
import jax.numpy as jnp
from jax import lax
from jax.experimental import pallas as pl
from jax.experimental.pallas import tpu as pltpu
from jax.experimental.pallas import tpu_sc as plsc

H = 9
NC, NS = 2, 16
NW = NC * NS
NEG = -1e30


def _mesh():
    return plsc.VectorSubcoreMesh(core_axis_name="c", subcore_axis_name="s")


def _perm(v, idx):
    return v.at[idx].get(mode="promise_in_bounds")


def _lanes_max(v):
    i = lax.iota(jnp.int32, 16)
    for sh in (8, 4, 2, 1):
        v = jnp.maximum(v, _perm(v, i ^ sh))
    return v


def _lanes_sum(v):
    i = lax.iota(jnp.int32, 16)
    for sh in (8, 4, 2, 1):
        v = v + _perm(v, i ^ sh)
    return v


def _pack_bf16_pair(a, b):
    ai = lax.bitcast_convert_type(a, jnp.int32)
    bi = lax.bitcast_convert_type(b, jnp.int32)
    ar = lax.shift_right_logical(
        ai + 0x7FFF + (lax.shift_right_logical(ai, 16) & 1), 16)
    br = lax.shift_right_logical(
        bi + 0x7FFF + (lax.shift_right_logical(bi, 16) & 1), 16)
    return ar | lax.shift_left(br, 16)


def _node_tables(xs, u_pad, wa, wb, cout, rows=1000):
    m = xs[0].shape[0]
    nx = len(xs)

    def body(*refs):
        x_refs = refs[:nx]
        u_refs = refs[nx:2 * nx]
        wa_refs = refs[2 * nx:3 * nx]
        wb_refs = refs[3 * nx:4 * nx]
        t_ref, xw_ref = refs[4 * nx:]
        t = jnp.dot(x_refs[0][...], u_refs[0][...],
                    preferred_element_type=jnp.float32)
        xa = jnp.dot(x_refs[0][...], wa_refs[0][...],
                     preferred_element_type=jnp.float32)
        xb = jnp.dot(x_refs[0][...], wb_refs[0][...],
                     preferred_element_type=jnp.float32)
        for i in range(1, nx):
            t += jnp.dot(x_refs[i][...], u_refs[i][...],
                         preferred_element_type=jnp.float32)
            xa += jnp.dot(x_refs[i][...], wa_refs[i][...],
                          preferred_element_type=jnp.float32)
            xb += jnp.dot(x_refs[i][...], wb_refs[i][...],
                          preferred_element_type=jnp.float32)
        t_ref[...] = t
        xw_ref[...] = _pack_bf16_pair(xa, xb)

    whalf = H * cout // 2
    in_specs = (
        [pl.BlockSpec((rows, x.shape[1]), lambda i: (i, 0)) for x in xs]
        + [pl.BlockSpec(uu.shape, lambda i: (0, 0)) for uu in u_pad]
        + [pl.BlockSpec(ww.shape, lambda i: (0, 0)) for ww in wa]
        + [pl.BlockSpec(ww.shape, lambda i: (0, 0)) for ww in wb]
    )
    out_specs = [pl.BlockSpec((rows, 16), lambda i: (i, 0)),
                 pl.BlockSpec((rows, whalf), lambda i: (i, 0))]
    return pl.pallas_call(
        body,
        grid=(m // rows,),
        in_specs=in_specs,
        out_specs=out_specs,
        out_shape=[jax.ShapeDtypeStruct((m, 16), jnp.float32),
                   jax.ShapeDtypeStruct((m, whalf), jnp.int32)],
    )(*xs, *u_pad, *wa, *wb)


def _epilogue(s, cnt, b, lrelu, rows=1000):
    m, cout = s.shape

    def body(s_ref, c_ref, b_ref, o_ref):
        cn = jnp.maximum(c_ref[:, 0:1], 1.0)
        y = s_ref[...] / cn + b_ref[...]
        if lrelu:
            y = jnp.where(y >= 0, y, 0.1 * y)
        o_ref[...] = y

    return pl.pallas_call(
        body,
        grid=(m // rows,),
        in_specs=[pl.BlockSpec((rows, cout), lambda i: (i, 0)),
                  pl.BlockSpec((rows, 16), lambda i: (i, 0)),
                  pl.BlockSpec((1, cout), lambda i: (0, 0))],
        out_specs=pl.BlockSpec((rows, cout), lambda i: (i, 0)),
        out_shape=jax.ShapeDtypeStruct((m, cout), jnp.float32),
    )(s, cnt, b.reshape(1, cout))


def _pair_max(y, rows=1000):
    m, c = y.shape
    y2 = y.reshape(m // 2, 2 * c)

    def body(a_ref, o_ref):
        a = a_ref[...]
        o_ref[...] = jnp.maximum(a[:, :c], a[:, c:])

    return pl.pallas_call(
        body,
        grid=(m // 2 // rows,),
        in_specs=[pl.BlockSpec((rows, 2 * c), lambda i: (i, 0))],
        out_specs=pl.BlockSpec((rows, c), lambda i: (i, 0)),
        out_shape=jax.ShapeDtypeStruct((m // 2, c), jnp.float32),
    )(y2)


def _fc_head(f, w1, b1, w2, b2, rows=1000):
    m = f.shape[0]

    def body(f_ref, w1_ref, b1_ref, w2_ref, b2_ref, o_ref):
        h1 = jnp.dot(f_ref[...], w1_ref[...],
                     preferred_element_type=jnp.float32) + b1_ref[...]
        h1 = jnp.where(h1 >= 0, h1, 0.1 * h1)
        o = jnp.dot(h1, w2_ref[...],
                    preferred_element_type=jnp.float32) + b2_ref[...]
        nrm = jnp.sqrt(jnp.sum(o * o, axis=1, keepdims=True))
        o_ref[...] = o / jnp.maximum(nrm, 1e-12)

    return pl.pallas_call(
        body,
        grid=(m // rows,),
        in_specs=[pl.BlockSpec((rows, f.shape[1]), lambda i: (i, 0)),
                  pl.BlockSpec(w1.shape, lambda i: (0, 0)),
                  pl.BlockSpec((1, b1.shape[0]), lambda i: (0, 0)),
                  pl.BlockSpec(w2.shape, lambda i: (0, 0)),
                  pl.BlockSpec((1, b2.shape[0]), lambda i: (0, 0))],
        out_specs=pl.BlockSpec((rows, w2.shape[1]), lambda i: (i, 0)),
        out_shape=jax.ShapeDtypeStruct((m, w2.shape[1]), jnp.float32),
    )(f, w1, b1.reshape(1, -1), w2, b2.reshape(1, -1))


def _shifted_indices(idx, shifts, cols=2000):
    e_tot = idx.shape[0]
    idx2 = idx.reshape(e_tot // cols, cols)

    def body(i_ref, *outs):
        v = i_ref[...]
        for s, o_ref in zip(shifts, outs):
            o_ref[...] = lax.shift_right_logical(v, s)

    rows = 80
    outs = pl.pallas_call(
        body,
        grid=(e_tot // cols // rows,),
        in_specs=[pl.BlockSpec((rows, cols), lambda i: (i, 0))],
        out_specs=[pl.BlockSpec((rows, cols), lambda i: (i, 0))
                   for _ in shifts],
        out_shape=[jax.ShapeDtypeStruct(idx2.shape, jnp.int32)
                   for _ in shifts],
    )(idx2)
    return [o.reshape(e_tot) for o in outs]


def _edge_msgs(gs_arr, gd_arr, t_tab, xw_tab, c16, cout, chunk):
    e_tot = gs_arr.shape[0]
    per_tile = e_tot // NW
    n_chunks = per_tile // chunk
    nb = cout // 16
    wrow = H * cout // 2

    def dbuf(shape, dt):
        return [pltpu.VMEM(shape, dt), pltpu.VMEM(shape, dt)]

    @functools.partial(
        pl.kernel,
        out_type=jax.ShapeDtypeStruct((e_tot, cout), jnp.float32),
        mesh=_mesh(),
        compiler_params=pltpu.CompilerParams(use_tc_tiling_on_sc=False),
        scratch_types=[
            dbuf((chunk,), jnp.int32),
            dbuf((chunk,), jnp.int32),
            dbuf((chunk, 16), jnp.float32),
            dbuf((chunk, 16), jnp.float32),
            dbuf((chunk, wrow), jnp.int32),
            dbuf((chunk, cout), jnp.float32),
            pltpu.VMEM((16,), jnp.float32),
            [pltpu.SemaphoreType.DMA, pltpu.SemaphoreType.DMA],
        ],
    )
    def k(gs_h, gd_h, t_h, xw_h, c_h, msg_h,
          gs_v, gd_v, ts_v, td_v, xw_v, msg_v, c_v, sem):
        wid = lax.axis_index("s") * NC + lax.axis_index("c")
        base0 = wid * per_tile
        pltpu.sync_copy(c_h, c_v)
        c16v = c_v[...]

        def prefetch(c, b):
            base = base0 + c * chunk
            pltpu.sync_copy(gs_h.at[pl.ds(base, chunk)], gs_v[b])
            pltpu.sync_copy(gd_h.at[pl.ds(base, chunk)], gd_v[b])
            pltpu.async_copy(t_h.at[gs_v[b]], ts_v[b], sem[b])
            pltpu.async_copy(t_h.at[gd_v[b]], td_v[b], sem[b])
            pltpu.async_copy(xw_h.at[gs_v[b]], xw_v[b], sem[b])

        def compute(c, b):
            pltpu.make_async_copy(t_h.at[gs_v[b]], ts_v[b], sem[b]).wait()
            pltpu.make_async_copy(t_h.at[gd_v[b]], td_v[b], sem[b]).wait()
            pltpu.make_async_copy(xw_h.at[gs_v[b]], xw_v[b], sem[b]).wait()

            def edge_body(e, _):
                d = ts_v[b][e, :] - td_v[b][e, :] + c16v
                ex = jnp.exp(d - _lanes_max(d))
                q = ex / _lanes_sum(ex)
                acc = [jnp.zeros((16,), jnp.float32) for _ in range(nb)]
                for hh in range(H):
                    qh = _perm(q, jnp.full((16,), hh, jnp.int32))
                    for g2 in range(nb // 2):
                        o = hh * (cout // 2) + g2 * 16
                        wi = xw_v[b][e, o:o + 16]
                        va = lax.bitcast_convert_type(
                            lax.shift_left(wi, 16), jnp.float32)
                        vb = lax.bitcast_convert_type(
                            wi & jnp.int32(-65536), jnp.float32)
                        acc[2 * g2] = acc[2 * g2] + qh * va
                        acc[2 * g2 + 1] = acc[2 * g2 + 1] + qh * vb
                for cb in range(nb):
                    msg_v[b][e, cb * 16:(cb + 1) * 16] = acc[cb]
                return 0

            lax.fori_loop(0, chunk, edge_body, 0)
            pltpu.sync_copy(msg_v[b], msg_h.at[pl.ds(base0 + c * chunk, chunk)])

        prefetch(0, 0)

        def pair_body(kk, _):
            c0 = 2 * kk
            prefetch(c0 + 1, 1)
            compute(c0, 0)

            @pl.when(c0 + 2 < n_chunks)
            def _():
                prefetch(c0 + 2, 0)

            compute(c0 + 1, 1)
            return 0

        lax.fori_loop(0, n_chunks // 2, pair_body, 0)
        if n_chunks % 2 == 1:
            compute(n_chunks - 1, 0)

    return k(gs_arr, gd_arr, t_tab, xw_tab, c16)


def _seg_sum(dst0, msg, d_shift, n, cout, chunk=80):
    e_tot = dst0.shape[0]
    half = n // 2
    rows = half + 100
    per_tile = e_tot // NS
    n_chunks = per_tile // chunk
    zc = rows // 100
    fc = half // 100

    def dbuf(shape, dt):
        return [pltpu.VMEM(shape, dt), pltpu.VMEM(shape, dt)]

    @functools.partial(
        pl.kernel,
        out_type=jax.ShapeDtypeStruct((n, cout), jnp.float32),
        mesh=_mesh(),
        compiler_params=pltpu.CompilerParams(use_tc_tiling_on_sc=False),
        scratch_types=[
            dbuf((chunk,), jnp.int32),
            dbuf((chunk,), jnp.int32),
            dbuf((chunk, cout), jnp.float32),
            pltpu.VMEM_SHARED((rows, cout), jnp.float32),
            [pltpu.SemaphoreType.DMA, pltpu.SemaphoreType.DMA],
        ],
    )
    def k(dst_h, msg_h, z_h, out_h, rd_v, idx_v, msg_v, acc, sem):
        cid = lax.axis_index("c")
        sid = lax.axis_index("s")
        node_base = cid * half
        base0 = sid * per_tile
        for zk in range((zc + NS - 1) // NS):
            ck = zk * NS + sid

            @pl.when(ck < zc)
            def _():
                pltpu.sync_copy(z_h, acc.at[pl.ds(ck * 100, 100)])
        plsc.subcore_barrier()

        def prefetch(c, b):
            base = base0 + c * chunk
            pltpu.sync_copy(dst_h.at[pl.ds(base, chunk)], rd_v[b])
            pltpu.async_copy(msg_h.at[pl.ds(base, chunk)], msg_v[b], sem[b])

        def consume(c, b):
            pltpu.make_async_copy(
                msg_h.at[pl.ds(base0 + c * chunk, chunk)], msg_v[b],
                sem[b]).wait()

            def shift_body(j, _):
                sl = pl.ds(j * 16, 16)
                loc = lax.shift_right_logical(rd_v[b][sl], d_shift) - node_base
                ok = (loc >= 0) & (loc < half)
                idx_v[b][sl] = jnp.where(ok, loc, half + sid)
                return 0

            lax.fori_loop(0, chunk // 16, shift_body, 0)
            pltpu.sync_copy(msg_v[b], acc.at[idx_v[b]], add=True)

        prefetch(0, 0)

        def pair_body(kk, _):
            c0 = 2 * kk
            prefetch(c0 + 1, 1)
            consume(c0, 0)

            @pl.when(c0 + 2 < n_chunks)
            def _():
                prefetch(c0 + 2, 0)

            consume(c0 + 1, 1)
            return 0

        lax.fori_loop(0, n_chunks // 2, pair_body, 0)
        if n_chunks % 2 == 1:
            consume(n_chunks - 1, 0)
        plsc.subcore_barrier()
        for fk in range((fc + NS - 1) // NS):
            ck = fk * NS + sid

            @pl.when(ck < fc)
            def _():
                pltpu.sync_copy(acc.at[pl.ds(ck * 100, 100)],
                                out_h.at[pl.ds(node_base + ck * 100, 100)])

    zeros = jnp.zeros((100, cout), jnp.float32)
    return k(dst0, msg, zeros)


def _counts(dst0, n1, chunk=80):
    e_tot = dst0.shape[0]
    halves = (n1 // 2, n1 // 4, n1 // 8)
    rows = tuple(h + 100 for h in halves)
    per_tile = e_tot // NS
    n_chunks = per_tile // chunk

    @functools.partial(
        pl.kernel,
        out_type=[jax.ShapeDtypeStruct((n1, 16), jnp.float32),
                  jax.ShapeDtypeStruct((n1 // 2, 16), jnp.float32),
                  jax.ShapeDtypeStruct((n1 // 4, 16), jnp.float32)],
        mesh=_mesh(),
        compiler_params=pltpu.CompilerParams(use_tc_tiling_on_sc=False),
        scratch_types=[
            pltpu.VMEM((chunk,), jnp.int32),
            pltpu.VMEM((chunk,), jnp.int32),
            pltpu.VMEM((chunk, 16), jnp.float32),
            pltpu.VMEM((100, 16), jnp.float32),
            pltpu.VMEM_SHARED((rows[0], 16), jnp.float32),
            pltpu.VMEM_SHARED((rows[1], 16), jnp.float32),
            pltpu.VMEM_SHARED((rows[2], 16), jnp.float32),
        ],
    )
    def k(dst_h, ones_h, z_h, c1_h, c2_h, c3_h,
          rd_v, idx_v, ones_v, z_v, acc1, acc2, acc3):
        cid = lax.axis_index("c")
        sid = lax.axis_index("s")
        pltpu.sync_copy(ones_h, ones_v)
        pltpu.sync_copy(z_h, z_v)
        for acc, rw in ((acc1, rows[0]), (acc2, rows[1]), (acc3, rows[2])):
            zc = rw // 100
            for zk in range((zc + NS - 1) // NS):
                ck = zk * NS + sid

                @pl.when(ck < zc)
                def _():
                    pltpu.sync_copy(z_v, acc.at[pl.ds(ck * 100, 100)])
        plsc.subcore_barrier()

        def chunk_body(kk, _):
            base = sid * per_tile + kk * chunk
            pltpu.sync_copy(dst_h.at[pl.ds(base, chunk)], rd_v)
            for lvl, (acc, hf) in enumerate(((acc1, halves[0]),
                                             (acc2, halves[1]),
                                             (acc3, halves[2]))):
                def shift_body(j, _, lvl=lvl, hf=hf):
                    sl = pl.ds(j * 16, 16)
                    loc = lax.shift_right_logical(rd_v[sl], lvl) - cid * hf
                    ok = (loc >= 0) & (loc < hf)
                    idx_v[sl] = jnp.where(ok, loc, hf + sid)
                    return 0

                lax.fori_loop(0, chunk // 16, shift_body, 0)
                pltpu.sync_copy(ones_v, acc.at[idx_v], add=True)
            return 0

        lax.fori_loop(0, n_chunks, chunk_body, 0)
        plsc.subcore_barrier()
        for acc, hf, out_h in ((acc1, halves[0], c1_h),
                               (acc2, halves[1], c2_h),
                               (acc3, halves[2], c3_h)):
            fc = hf // 100
            for fk in range((fc + NS - 1) // NS):
                ck = fk * NS + sid

                @pl.when(ck < fc)
                def _():
                    pltpu.sync_copy(acc.at[pl.ds(ck * 100, 100)],
                                    out_h.at[pl.ds(cid * hf + ck * 100, 100)])

    ones = jnp.zeros((chunk, 16), jnp.float32).at[:, 0].set(1.0)
    zeros = jnp.zeros((100, 16), jnp.float32)
    return k(dst0, ones, zeros)


def _pad_u(u):
    cin = u.shape[0]
    cp = max(8, cin)
    out = jnp.zeros((cp, 16), jnp.float32)
    return out.at[:cin, :H].set(u)


def _pad_w(w):
    cin = w.shape[0]
    cp = max(8, cin)
    if cp == cin:
        return w
    return jnp.zeros((cp, w.shape[1]), jnp.float32).at[:cin].set(w)


def _c16(c):
    return jnp.concatenate([c, jnp.full((16 - H,), NEG, jnp.float32)])


def _feast(xs, gs_arr, gd_arr, dst0, cnt, p, d_shift, n_seg, cout, lrelu):
    u_parts, wa_parts, wb_parts, x_parts = [], [], [], []
    off = 0
    for xx in xs:
        cw = xx.shape[1]
        u_parts.append(_pad_u(p["u"][off:off + cw]))
        wp = _pad_w(p["W"][off:off + cw])
        w3 = wp.reshape(wp.shape[0], -1, 32)
        wa_parts.append(w3[:, :, :16].reshape(wp.shape[0], -1))
        wb_parts.append(w3[:, :, 16:].reshape(wp.shape[0], -1))
        x_parts.append(xx if cw >= 8
                       else jnp.pad(xx, ((0, 0), (0, 8 - cw))))
        off += cw
    t_tab, xw_tab = _node_tables(x_parts, u_parts, wa_parts, wb_parts, cout)
    chunk = 80
    msg = _edge_msgs(gs_arr, gd_arr, t_tab, xw_tab, _c16(p["c"]), cout, chunk)
    s = _seg_sum(dst0, msg, d_shift, n_seg, cout)
    return _epilogue(s, cnt, p["b"], lrelu)


def kernel(x, edge_index, params):
    p = params
    src0 = edge_index[0]
    dst0 = edge_index[1]
    n1 = x.shape[0]

    s1, s2 = _shifted_indices(src0, (1, 2))
    d1, d2 = _shifted_indices(dst0, (1, 2))
    cnt1, cnt2, cnt3 = _counts(dst0, n1)

    x1 = _feast([x], src0, dst0, dst0, cnt1, p["l1"], 0, n1, 32, True)
    x2p = _pair_max(x1)
    x2 = _feast([x2p], s1, d1, dst0, cnt2, p["l2"], 1, n1 // 2, 64, True)
    x3p = _pair_max(x2)
    x3 = _feast([x3p], s2, d2, dst0, cnt3, p["l3"], 2, n1 // 4, 128, True)
    x3 = _feast([x3], s2, d2, dst0, cnt3, p["l4"], 2, n1 // 4, 128, True)
    f2 = _feast([x3], s2, d2, dst0, cnt2, p["r1"], 1, n1 // 2, 64, False)
    y2 = _feast([x2, f2], s1, d1, dst0, cnt2, p["r2"], 1, n1 // 2, 64, True)
    f1 = _feast([y2], s1, d1, dst0, cnt1, p["r3"], 0, n1, 32, False)
    f = _feast([x1, f1], src0, dst0, dst0, cnt1, p["r4"], 0, n1, 32, True)
    return _fc_head(f, p["fc1w"], p["fc1b"], p["fc2w"], p["fc2b"])
